# Initial kernel scaffold; baseline (speedup 1.0000x reference)
import jax
import jax.numpy as jnp
from jax.experimental import pallas as pl


def kernel(primary_data, secondary_data, rule_vecs, params, gumbel1, gumbel2):
    raise NotImplementedError("write your pallas kernel here")


# speedup vs baseline: 1.1443x; 1.1443x over previous
import jax
import jax.numpy as jnp
from jax import lax
from jax.experimental import pallas as pl
from jax.experimental.pallas import tpu as pltpu

_NP = 8192
_NS = 8192
_R = 4
_SL = 16
_MAXI = 2147483647


def _dn(a, b):
    return lax.dot_general(a, b, (((1,), (0,)), ((), ())))


def _dnt(a, b):
    return lax.dot_general(a, b, (((1,), (1,)), ((), ())))


def _gnoise(u):
    return -jnp.log(-jnp.log(u + 1e-20) + 1e-20)


def _bf(x):
    return x.astype(jnp.bfloat16).astype(jnp.float32)


def _body(primary, secondary, rule_vecs, g1t, g2r,
          Wq1, bq1, Wq2, bq2, Wk1, bk1, Wk2, bk2,
          Wqn1, bqn1, Wqn2, bqn2, Wkn1, bkn1, Wkn2, bkn2,
          rW1, rb1, rW2, rb2, pW1, pb1, pW2, pb2,
          o_ps, o_ss, o_rm, o_po, o_ap, o_pc, sq2_scr):
    h = jnp.maximum(_dn(primary[...], Wq1[...]) + bq1[...], 0.0)
    sq = _dn(h, Wq2[...]) + bq2[...]
    hk = jnp.maximum(_dn(rule_vecs[...], Wk1[...]) + bk1[...], 0.0)
    rk = _dn(hk, Wk2[...]) + bk2[...]
    s1 = _dnt(rk, sq) + _gnoise(g1t[...])
    m1 = jnp.max(s1)
    fi = (lax.broadcasted_iota(jnp.int32, (_R, _NP), 1) * _R
          + lax.broadcasted_iota(jnp.int32, (_R, _NP), 0))
    flat1 = jnp.min(jnp.where(s1 == m1, fi, _MAXI))
    i_star = flat1 // _R
    r_star = flat1 - i_star * _R

    hq = jnp.maximum(_dn(primary[...], Wqn1[...]) + bqn1[...], 0.0)
    sq2_scr[...] = _dn(hq, Wqn2[...]) + bqn2[...]
    q = _bf(sq2_scr[pl.ds(i_star, 1), :])
    hs = jnp.maximum(_dn(secondary[...], Wkn1[...]) + bkn1[...], 0.0)
    sk = _dn(hs, Wkn2[...]) + bkn2[...]
    s2 = _dnt(q, sk) + _gnoise(g2r[...])
    m2 = jnp.max(s2)
    ji = lax.broadcasted_iota(jnp.int32, (1, _NS), 1)
    j_star = jnp.min(jnp.where(s2 == m2, ji, _MAXI))

    prow = _bf(primary[pl.ds(i_star, 1), :])
    srow = _bf(secondary[pl.ds(j_star, 1), :])
    o_ps[...] = prow
    o_ss[...] = srow
    rm = (lax.broadcasted_iota(jnp.int32, (1, _R), 1) == r_star
          ).astype(jnp.float32)
    o_rm[...] = rm
    ps2 = prow[:, 0:2]
    rule_in = jnp.concatenate([ps2, ps2], axis=1)
    ap_rows = []
    for r in range(_R):
        hr = jnp.maximum(_dn(rule_in, rW1[r]) + rb1[r:r + 1, :], 0.0)
        ap_rows.append(_dn(hr, rW2[r]) + rb2[r:r + 1, :])
    ap = jnp.concatenate(ap_rows, axis=0)
    o_ap[...] = ap
    sel = (lax.broadcasted_iota(jnp.int32, (_R, 1), 0) == r_star
           ).astype(jnp.float32)
    o_po[...] = jnp.sum(_bf(ap) * sel, axis=0, keepdims=True)
    pin = jnp.concatenate([ps2, srow[:, 0:2], rm], axis=1)
    hp = jnp.maximum(_dn(pin, pW1[...]) + pb1[...], 0.0)
    o_pc[...] = _dn(hp, pW2[...]) + pb2[...]


def kernel(primary_data, secondary_data, rule_vecs, params, gumbel1, gumbel2):
    p = params
    g1t = gumbel1.reshape(_NP, _R).T
    g2r = gumbel2.reshape(1, _NS)
    args = (
        primary_data, secondary_data, rule_vecs, g1t, g2r,
        p['Wq1'], p['bq1'].reshape(1, -1), p['Wq2'], p['bq2'].reshape(1, -1),
        p['Wk1'], p['bk1'].reshape(1, -1), p['Wk2'], p['bk2'].reshape(1, -1),
        p['Wqn1'], p['bqn1'].reshape(1, -1), p['Wqn2'], p['bqn2'].reshape(1, -1),
        p['Wkn1'], p['bkn1'].reshape(1, -1), p['Wkn2'], p['bkn2'].reshape(1, -1),
        p['rW1'], p['rb1'], p['rW2'], p['rb2'],
        p['pW1'], p['pb1'].reshape(1, -1), p['pW2'], p['pb2'].reshape(1, -1),
    )
    o_ps, o_ss, o_rm, o_po, o_ap, o_pc = pl.pallas_call(
        _body,
        out_shape=[
            jax.ShapeDtypeStruct((1, 8), jnp.float32),
            jax.ShapeDtypeStruct((1, 8), jnp.float32),
            jax.ShapeDtypeStruct((1, _R), jnp.float32),
            jax.ShapeDtypeStruct((1, 2), jnp.float32),
            jax.ShapeDtypeStruct((_R, 2), jnp.float32),
            jax.ShapeDtypeStruct((1, 1), jnp.float32),
        ],
        scratch_shapes=[pltpu.VMEM((_NP, _SL), jnp.float32)],
    )(*args)
    return (o_ps[0], o_ss[0], o_rm[0], o_po[0], o_ap, o_pc[0, 0])
